# baseline (speedup 1.0000x reference)
import functools

import jax
import jax.numpy as jnp
from jax.experimental import pallas as pl
from jax.experimental.pallas import tpu as pltpu

_LANE = 128
_VMEM_LIMIT_BYTES = 48 * 1024 * 1024

_RESIDENT = pl.Buffered(1)


def _cparams(dimension_semantics):
    return pltpu.CompilerParams(
        dimension_semantics=dimension_semantics,
        vmem_limit_bytes=_VMEM_LIMIT_BYTES,
    )


def _trunk_kernel(xs_ref, w_ref, b_ref, o_ref, *, IPB, H, W):
    Kp = xs_ref.shape[-1]
    H2, W2 = H // 2, W // 2
    m = IPB * H * W
    patch = xs_ref[...].reshape(m, Kp)
    out = jnp.dot(patch, w_ref[...], preferred_element_type=jnp.float32)
    out = jnp.maximum(out + b_ref[...], 0.0)
    C = out.shape[-1]
    r = out.reshape(IPB, 2, 2, H2, W2, C).max(axis=2).max(axis=1)
    o_ref[...] = r.astype(o_ref.dtype)


def _trunk(x_nchw, w_oikk, b):
    N, Cin, H, W = x_nchw.shape
    Cout, _, k, _ = w_oikk.shape
    pad = k // 2
    H2, W2 = H // 2, W // 2
    xs = jax.lax.conv_general_dilated_patches(
        x_nchw, (k, k), (1, 1), ((pad, pad), (pad, pad)))
    K = xs.shape[1]
    Kp = K + ((-K) % 32)
    xs = jnp.pad(xs.astype(jnp.bfloat16), ((0, 0), (0, Kp - K), (0, 0), (0, 0)))
    xs = xs.reshape(N, Kp, H2, 2, W2, 2).transpose(0, 3, 5, 2, 4, 1)
    w = w_oikk.reshape(Cout, K).T
    w = jnp.pad(w, ((0, Kp - K), (0, 0))).astype(jnp.bfloat16)
    b2 = b.reshape(1, Cout).astype(jnp.float32)

    IPB = 4
    cost = pl.CostEstimate(
        flops=2 * N * H * W * K * Cout,
        transcendentals=0,
        bytes_accessed=xs.size * 2 + w.size * 2 + b2.size * 4
        + N * H2 * W2 * Cout * 2,
    )
    fn = functools.partial(_trunk_kernel, IPB=IPB, H=H, W=W)
    return pl.pallas_call(
        fn,
        out_shape=jax.ShapeDtypeStruct((N, H2, W2, Cout), jnp.bfloat16),
        grid_spec=pltpu.PrefetchScalarGridSpec(
            num_scalar_prefetch=0,
            grid=(N // IPB,),
            in_specs=[
                pl.BlockSpec((IPB, 2, 2, H2, W2, Kp), lambda n: (n, 0, 0, 0, 0, 0)),
                pl.BlockSpec((Kp, Cout), lambda n: (0, 0), pipeline_mode=_RESIDENT),
                pl.BlockSpec((1, Cout), lambda n: (0, 0), pipeline_mode=_RESIDENT),
            ],
            out_specs=pl.BlockSpec((IPB, H2, W2, Cout), lambda n: (n, 0, 0, 0)),
        ),
        compiler_params=_cparams(("parallel",)),
        cost_estimate=cost,
    )(xs, w, b2)


def _interp_mat(n_in, n_out):
    src = jnp.arange(n_out, dtype=jnp.float32) * (n_in - 1) / (n_out - 1)
    i0 = jnp.clip(jnp.floor(src).astype(jnp.int32), 0, n_in - 2)
    frac = src - i0.astype(jnp.float32)
    m = jnp.zeros((n_out, n_in), jnp.float32)
    m = m.at[jnp.arange(n_out), i0].add(1.0 - frac)
    m = m.at[jnp.arange(n_out), i0 + 1].add(frac)
    return m


def _upsample2x(x):
    N, H, W, C = x.shape
    Ah = _interp_mat(H, 2 * H)
    Aw = _interp_mat(W, 2 * W)
    y = jnp.einsum("oh,nhwc,pw->nopc", Ah, x.astype(jnp.float32), Aw)
    return y.astype(x.dtype)


def _heads_kernel(x_ref, wb_ref, w3_ref, bias_ref, ow_ref, ob_ref,
                  o_ref, xsrc_ref, xs_ref, accb_ref, *, H, W, Cin, IPB):
    HW = H * W
    K7 = 7 * Cin
    M = IPB * HW

    xsrc_ref[...] = jnp.zeros_like(xsrc_ref)
    xsrc_ref[3:3 + H, 3:3 + W, :] = x_ref[0]

    for kx in range(7):
        xs_ref[:, :, kx * Cin:(kx + 1) * Cin] = xsrc_ref[:, kx:kx + W, :]

    for i, ky in enumerate((0, 1, 5, 6)):
        patch = xs_ref[ky:ky + H].reshape(M, K7)
        d = jnp.dot(patch, wb_ref[i], preferred_element_type=jnp.float32)
        if ky == 0:
            accb_ref[:, _LANE:] = d
        else:
            accb_ref[:, _LANE:] += d
    for ky in (2, 3, 4):
        patch = xs_ref[ky:ky + H].reshape(M, K7)
        d = jnp.dot(patch, w3_ref[ky - 2], preferred_element_type=jnp.float32)
        if ky == 2:
            accb_ref[:, :_LANE] = d[:, :_LANE]
            accb_ref[:, _LANE:] += d[:, _LANE:]
        else:
            accb_ref[...] += d

    z = jnp.maximum(accb_ref[...] + bias_ref[...], 0.0)
    y = jnp.dot(ow_ref[...], z.T, preferred_element_type=jnp.float32)
    o_ref[...] = jnp.abs(y[0:1, :] + ob_ref[0]).reshape(1, 1, M)


def _heads(x_up, w3_, b3, w5, b5, w7, b7, ow, ob):
    N, H, W, Cin = x_up.shape
    k, pad = 7, 3
    HW = H * W
    CT = 3 * _LANE

    w5p = jnp.pad(w5, ((0, 0), (0, 0), (1, 1), (1, 1)))
    wbf = jnp.concatenate([w5p, w7], axis=0)
    wbf = jnp.transpose(wbf, (2, 3, 1, 0)).reshape(7, 7 * Cin, 2 * _LANE)
    wbf = wbf.astype(jnp.bfloat16)
    wb = wbf[jnp.array([0, 1, 5, 6])]
    w3p = jnp.pad(w3_, ((0, 0), (0, 0), (2, 2), (2, 2)))
    w3p = jnp.transpose(w3p, (2, 3, 1, 0)).reshape(7, 7 * Cin, _LANE)
    w3m = jnp.concatenate([w3p[2:5], wbf[2:5]], axis=-1)
    w3m = w3m.astype(jnp.bfloat16)
    bf = jnp.concatenate([b3, b5, b7], axis=0).reshape(1, CT).astype(jnp.float32)
    ow8 = jnp.broadcast_to(ow.reshape(1, CT).astype(jnp.float32), (8, CT))
    ob1 = ob.reshape(1).astype(jnp.float32)

    Hp, Wp, K7 = H + 2 * pad, W + 2 * pad + 2, k * Cin

    flops = (2 * N * HW * (49 * Cin * 2 * _LANE + 9 * Cin * _LANE)
             + 2 * N * HW * CT)
    cost = pl.CostEstimate(
        flops=flops,
        transcendentals=0,
        bytes_accessed=(x_up.size * 2 + wb.size * 2 + w3m.size * 2
                        + bf.size * 4 + ow8.size * 4 + N * HW * 4),
    )
    IPB = 1
    fn = functools.partial(_heads_kernel, H=H, W=W, Cin=Cin, IPB=IPB)
    out = pl.pallas_call(
        fn,
        out_shape=jax.ShapeDtypeStruct((N // IPB, 1, IPB * HW), jnp.float32),
        grid_spec=pltpu.PrefetchScalarGridSpec(
            num_scalar_prefetch=0,
            grid=(N // IPB,),
            in_specs=[
                pl.BlockSpec((IPB, H, W, Cin), lambda n: (n, 0, 0, 0)),
                pl.BlockSpec((4, K7, 2 * _LANE), lambda n: (0, 0, 0),
                             pipeline_mode=_RESIDENT),
                pl.BlockSpec((3, K7, 3 * _LANE), lambda n: (0, 0, 0),
                             pipeline_mode=_RESIDENT),
                pl.BlockSpec((1, CT), lambda n: (0, 0), pipeline_mode=_RESIDENT),
                pl.BlockSpec((8, CT), lambda n: (0, 0), pipeline_mode=_RESIDENT),
                pl.BlockSpec(memory_space=pltpu.MemorySpace.SMEM),
            ],
            out_specs=pl.BlockSpec((1, 1, IPB * HW), lambda n: (n, 0, 0)),
            scratch_shapes=[
                pltpu.VMEM((Hp, Wp, Cin), jnp.bfloat16),
                pltpu.VMEM((Hp, IPB * W, K7), jnp.bfloat16),
                pltpu.VMEM((IPB * HW, 3 * _LANE), jnp.float32),
            ],
        ),
        compiler_params=_cparams(("parallel",)),
        cost_estimate=cost,
    )(x_up, wb, w3m, bf, ow8, ob1)
    out = out.reshape(N // IPB, H, IPB, W).transpose(0, 2, 1, 3)
    return out.reshape(N, 1, H, W)


def kernel(x_nchw, features_w, features_b, reg3_w, reg3_b, reg5_w, reg5_b,
           reg7_w, reg7_b, out_w, out_b):
    x = _trunk(x_nchw, features_w, features_b)
    x = _upsample2x(x)
    return _heads(x, reg3_w, reg3_b, reg5_w, reg5_b, reg7_w, reg7_b,
                  out_w, out_b)


# speedup vs baseline: 1.1979x; 1.1979x over previous
import functools

import jax
import jax.numpy as jnp
from jax.experimental import pallas as pl
from jax.experimental.pallas import tpu as pltpu

_LANE = 128
_VMEM_LIMIT_BYTES = 48 * 1024 * 1024

_RESIDENT = pl.Buffered(1)


def _cparams(dimension_semantics):
    return pltpu.CompilerParams(
        dimension_semantics=dimension_semantics,
        vmem_limit_bytes=_VMEM_LIMIT_BYTES,
    )


def _trunk_kernel(xs_ref, w_ref, b_ref, o_ref, *, IPB, H, W):
    Kp = xs_ref.shape[-1]
    H2, W2 = H // 2, W // 2
    m = IPB * H * W
    patch = xs_ref[...].reshape(m, Kp)
    out = jnp.dot(patch, w_ref[...], preferred_element_type=jnp.float32)
    out = jnp.maximum(out + b_ref[...], 0.0)
    C = out.shape[-1]
    r = out.reshape(IPB, 2, 2, H2, W2, C).max(axis=2).max(axis=1)
    o_ref[...] = r.astype(o_ref.dtype)


def _trunk(x_nchw, w_oikk, b):
    N, Cin, H, W = x_nchw.shape
    Cout, _, k, _ = w_oikk.shape
    pad = k // 2
    H2, W2 = H // 2, W // 2
    xs = jax.lax.conv_general_dilated_patches(
        x_nchw, (k, k), (1, 1), ((pad, pad), (pad, pad)))
    K = xs.shape[1]
    Kp = K + ((-K) % 32)
    xs = jnp.pad(xs.astype(jnp.bfloat16), ((0, 0), (0, Kp - K), (0, 0), (0, 0)))
    xs = xs.reshape(N, Kp, H2, 2, W2, 2).transpose(0, 3, 5, 2, 4, 1)
    w = w_oikk.reshape(Cout, K).T
    w = jnp.pad(w, ((0, Kp - K), (0, 0))).astype(jnp.bfloat16)
    b2 = b.reshape(1, Cout).astype(jnp.float32)

    IPB = 4
    cost = pl.CostEstimate(
        flops=2 * N * H * W * K * Cout,
        transcendentals=0,
        bytes_accessed=xs.size * 2 + w.size * 2 + b2.size * 4
        + N * H2 * W2 * Cout * 2,
    )
    fn = functools.partial(_trunk_kernel, IPB=IPB, H=H, W=W)
    return pl.pallas_call(
        fn,
        out_shape=jax.ShapeDtypeStruct((N, H2, W2, Cout), jnp.bfloat16),
        grid_spec=pltpu.PrefetchScalarGridSpec(
            num_scalar_prefetch=0,
            grid=(N // IPB,),
            in_specs=[
                pl.BlockSpec((IPB, 2, 2, H2, W2, Kp), lambda n: (n, 0, 0, 0, 0, 0)),
                pl.BlockSpec((Kp, Cout), lambda n: (0, 0), pipeline_mode=_RESIDENT),
                pl.BlockSpec((1, Cout), lambda n: (0, 0), pipeline_mode=_RESIDENT),
            ],
            out_specs=pl.BlockSpec((IPB, H2, W2, Cout), lambda n: (n, 0, 0, 0)),
        ),
        compiler_params=_cparams(("parallel",)),
        cost_estimate=cost,
    )(xs, w, b2)


def _interp_mat(n_in, n_out):
    src = jnp.arange(n_out, dtype=jnp.float32) * (n_in - 1) / (n_out - 1)
    i0 = jnp.clip(jnp.floor(src).astype(jnp.int32), 0, n_in - 2)
    frac = src - i0.astype(jnp.float32)
    m = jnp.zeros((n_out, n_in), jnp.float32)
    m = m.at[jnp.arange(n_out), i0].add(1.0 - frac)
    m = m.at[jnp.arange(n_out), i0 + 1].add(frac)
    return m


def _upsample2x(x):
    N, H, W, C = x.shape
    Ah = _interp_mat(H, 2 * H)
    Aw = _interp_mat(W, 2 * W)
    y = jnp.einsum("oh,nhwc,pw->nopc", Ah, x.astype(jnp.float32), Aw)
    return y.astype(x.dtype)


def _heads_kernel(x_ref, wb_ref, w3_ref, bias_ref, ow_ref, ob_ref,
                  o_ref, xsrc_ref, xs_ref, accb_ref, acc3_ref, *, H, W, Cin, IPB):
    HW = H * W
    K7 = 7 * Cin
    M = IPB * HW

    xsrc_ref[...] = jnp.zeros_like(xsrc_ref)
    xsrc_ref[3:3 + H, 3:3 + W, :] = x_ref[0]

    for kx in range(7):
        xs_ref[:, :, kx * Cin:(kx + 1) * Cin] = xsrc_ref[:, kx:kx + W, :]

    for ky in range(7):
        patch = xs_ref[ky:ky + H].reshape(M, K7)
        d = jnp.dot(patch, wb_ref[ky], preferred_element_type=jnp.float32)
        if ky == 0:
            accb_ref[...] = d
        else:
            accb_ref[...] += d

    for ky in range(2, 5):
        p3 = xs_ref[ky:ky + H, :, 2 * Cin:5 * Cin].reshape(M, 3 * Cin)
        d = jnp.dot(p3, w3_ref[ky - 2], preferred_element_type=jnp.float32)
        if ky == 2:
            acc3_ref[...] = d
        else:
            acc3_ref[...] += d

    z = jnp.concatenate([acc3_ref[...], accb_ref[...]], axis=-1)
    z = jnp.maximum(z + bias_ref[...], 0.0)
    y = jnp.dot(ow_ref[...], z.T, preferred_element_type=jnp.float32)
    o_ref[...] = jnp.abs(y[0:1, :] + ob_ref[0]).reshape(1, 1, M)


def _heads(x_up, w3_, b3, w5, b5, w7, b7, ow, ob):
    N, H, W, Cin = x_up.shape
    k, pad = 7, 3
    HW = H * W
    CT = 3 * _LANE

    w5p = jnp.pad(w5, ((0, 0), (0, 0), (1, 1), (1, 1)))
    wb = jnp.concatenate([w5p, w7], axis=0)
    wb = jnp.transpose(wb, (2, 3, 1, 0)).reshape(7, 7 * Cin, 2 * _LANE)
    wb = wb.astype(jnp.bfloat16)
    w3m = jnp.transpose(w3_, (2, 3, 1, 0)).reshape(3, 3 * Cin, _LANE)
    w3m = w3m.astype(jnp.bfloat16)
    bf = jnp.concatenate([b3, b5, b7], axis=0).reshape(1, CT).astype(jnp.float32)
    ow8 = jnp.broadcast_to(ow.reshape(1, CT).astype(jnp.float32), (8, CT))
    ob1 = ob.reshape(1).astype(jnp.float32)

    Hp, Wp, K7 = H + 2 * pad, W + 2 * pad + 2, k * Cin

    flops = (2 * N * HW * (49 * Cin * 2 * _LANE + 9 * Cin * _LANE)
             + 2 * N * HW * CT)
    cost = pl.CostEstimate(
        flops=flops,
        transcendentals=0,
        bytes_accessed=(x_up.size * 2 + wb.size * 2 + w3m.size * 2
                        + bf.size * 4 + ow8.size * 4 + N * HW * 4),
    )
    IPB = 1
    fn = functools.partial(_heads_kernel, H=H, W=W, Cin=Cin, IPB=IPB)
    out = pl.pallas_call(
        fn,
        out_shape=jax.ShapeDtypeStruct((N // IPB, 1, IPB * HW), jnp.float32),
        grid_spec=pltpu.PrefetchScalarGridSpec(
            num_scalar_prefetch=0,
            grid=(N // IPB,),
            in_specs=[
                pl.BlockSpec((IPB, H, W, Cin), lambda n: (n, 0, 0, 0)),
                pl.BlockSpec((7, K7, 2 * _LANE), lambda n: (0, 0, 0),
                             pipeline_mode=_RESIDENT),
                pl.BlockSpec((3, 3 * Cin, _LANE), lambda n: (0, 0, 0),
                             pipeline_mode=_RESIDENT),
                pl.BlockSpec((1, CT), lambda n: (0, 0), pipeline_mode=_RESIDENT),
                pl.BlockSpec((8, CT), lambda n: (0, 0), pipeline_mode=_RESIDENT),
                pl.BlockSpec(memory_space=pltpu.MemorySpace.SMEM),
            ],
            out_specs=pl.BlockSpec((1, 1, IPB * HW), lambda n: (n, 0, 0)),
            scratch_shapes=[
                pltpu.VMEM((Hp, Wp, Cin), jnp.bfloat16),
                pltpu.VMEM((Hp, IPB * W, K7), jnp.bfloat16),
                pltpu.VMEM((IPB * HW, 2 * _LANE), jnp.float32),
                pltpu.VMEM((IPB * HW, _LANE), jnp.float32),
            ],
        ),
        compiler_params=_cparams(("parallel",)),
        cost_estimate=cost,
    )(x_up, wb, w3m, bf, ow8, ob1)
    out = out.reshape(N // IPB, H, IPB, W).transpose(0, 2, 1, 3)
    return out.reshape(N, 1, H, W)


def kernel(x_nchw, features_w, features_b, reg3_w, reg3_b, reg5_w, reg5_b,
           reg7_w, reg7_b, out_w, out_b):
    x = _trunk(x_nchw, features_w, features_b)
    x = _upsample2x(x)
    return _heads(x, reg3_w, reg3_b, reg5_w, reg5_b, reg7_w, reg7_b,
                  out_w, out_b)


# speedup vs baseline: 1.2003x; 1.0020x over previous
import functools

import jax
import jax.numpy as jnp
from jax.experimental import pallas as pl
from jax.experimental.pallas import tpu as pltpu

_LANE = 128
_VMEM_LIMIT_BYTES = 48 * 1024 * 1024

_RESIDENT = pl.Buffered(1)


def _cparams(dimension_semantics):
    return pltpu.CompilerParams(
        dimension_semantics=dimension_semantics,
        vmem_limit_bytes=_VMEM_LIMIT_BYTES,
    )


def _trunk_kernel(xs_ref, w_ref, b_ref, o_ref, *, IPB, H, W):
    Kp = xs_ref.shape[-1]
    H2, W2 = H // 2, W // 2
    m = IPB * H * W
    patch = xs_ref[...].reshape(m, Kp)
    out = jnp.dot(patch, w_ref[...], preferred_element_type=jnp.float32)
    out = jnp.maximum(out + b_ref[...], 0.0)
    C = out.shape[-1]
    r = out.reshape(IPB, 2, 2, H2, W2, C).max(axis=2).max(axis=1)
    o_ref[...] = r.astype(o_ref.dtype)


def _trunk(x_nchw, w_oikk, b):
    N, Cin, H, W = x_nchw.shape
    Cout, _, k, _ = w_oikk.shape
    pad = k // 2
    H2, W2 = H // 2, W // 2
    xs = jax.lax.conv_general_dilated_patches(
        x_nchw, (k, k), (1, 1), ((pad, pad), (pad, pad)))
    K = xs.shape[1]
    Kp = K + ((-K) % 32)
    xs = jnp.pad(xs.astype(jnp.bfloat16), ((0, 0), (0, Kp - K), (0, 0), (0, 0)))
    xs = xs.reshape(N, Kp, H2, 2, W2, 2).transpose(0, 3, 5, 2, 4, 1)
    w = w_oikk.reshape(Cout, K).T
    w = jnp.pad(w, ((0, Kp - K), (0, 0))).astype(jnp.bfloat16)
    b2 = b.reshape(1, Cout).astype(jnp.float32)

    IPB = 8
    cost = pl.CostEstimate(
        flops=2 * N * H * W * K * Cout,
        transcendentals=0,
        bytes_accessed=xs.size * 2 + w.size * 2 + b2.size * 4
        + N * H2 * W2 * Cout * 2,
    )
    fn = functools.partial(_trunk_kernel, IPB=IPB, H=H, W=W)
    return pl.pallas_call(
        fn,
        out_shape=jax.ShapeDtypeStruct((N, H2, W2, Cout), jnp.bfloat16),
        grid_spec=pltpu.PrefetchScalarGridSpec(
            num_scalar_prefetch=0,
            grid=(N // IPB,),
            in_specs=[
                pl.BlockSpec((IPB, 2, 2, H2, W2, Kp), lambda n: (n, 0, 0, 0, 0, 0)),
                pl.BlockSpec((Kp, Cout), lambda n: (0, 0), pipeline_mode=_RESIDENT),
                pl.BlockSpec((1, Cout), lambda n: (0, 0), pipeline_mode=_RESIDENT),
            ],
            out_specs=pl.BlockSpec((IPB, H2, W2, Cout), lambda n: (n, 0, 0, 0)),
        ),
        compiler_params=_cparams(("parallel",)),
        cost_estimate=cost,
    )(xs, w, b2)


def _interp_mat(n_in, n_out):
    src = jnp.arange(n_out, dtype=jnp.float32) * (n_in - 1) / (n_out - 1)
    i0 = jnp.clip(jnp.floor(src).astype(jnp.int32), 0, n_in - 2)
    frac = src - i0.astype(jnp.float32)
    m = jnp.zeros((n_out, n_in), jnp.float32)
    m = m.at[jnp.arange(n_out), i0].add(1.0 - frac)
    m = m.at[jnp.arange(n_out), i0 + 1].add(frac)
    return m


def _upsample2x(x):
    N, H, W, C = x.shape
    Ah = _interp_mat(H, 2 * H)
    Aw = _interp_mat(W, 2 * W)
    y = jnp.einsum("oh,nhwc,pw->nopc", Ah, x.astype(jnp.float32), Aw)
    return y.astype(x.dtype)


def _heads_kernel(x_ref, wb_ref, w3_ref, bias_ref, ow_ref, ob_ref,
                  o_ref, xsrc_ref, xs_ref, accb_ref, acc3_ref, *, H, W, Cin, IPB):
    HW = H * W
    K7 = 7 * Cin
    M = IPB * HW

    xsrc_ref[...] = jnp.zeros_like(xsrc_ref)
    xsrc_ref[3:3 + H, 3:3 + W, :] = x_ref[0]

    for kx in range(7):
        xs_ref[:, :, kx * Cin:(kx + 1) * Cin] = xsrc_ref[:, kx:kx + W, :]

    for ky in range(7):
        patch = xs_ref[ky:ky + H].reshape(M, K7)
        d = jnp.dot(patch, wb_ref[ky], preferred_element_type=jnp.float32)
        if ky == 0:
            accb_ref[...] = d
        else:
            accb_ref[...] += d

    for ky in range(2, 5):
        p3 = xs_ref[ky:ky + H, :, 2 * Cin:5 * Cin].reshape(M, 3 * Cin)
        d = jnp.dot(p3, w3_ref[ky - 2], preferred_element_type=jnp.float32)
        if ky == 2:
            acc3_ref[...] = d
        else:
            acc3_ref[...] += d

    z = jnp.concatenate([acc3_ref[...], accb_ref[...]], axis=-1)
    z = jnp.maximum(z + bias_ref[...], 0.0)
    y = jnp.dot(ow_ref[...], z.T, preferred_element_type=jnp.float32)
    o_ref[...] = jnp.abs(y[0:1, :] + ob_ref[0]).reshape(1, 1, M)


def _heads(x_up, w3_, b3, w5, b5, w7, b7, ow, ob):
    N, H, W, Cin = x_up.shape
    k, pad = 7, 3
    HW = H * W
    CT = 3 * _LANE

    w5p = jnp.pad(w5, ((0, 0), (0, 0), (1, 1), (1, 1)))
    wb = jnp.concatenate([w5p, w7], axis=0)
    wb = jnp.transpose(wb, (2, 3, 1, 0)).reshape(7, 7 * Cin, 2 * _LANE)
    wb = wb.astype(jnp.bfloat16)
    w3m = jnp.transpose(w3_, (2, 3, 1, 0)).reshape(3, 3 * Cin, _LANE)
    w3m = w3m.astype(jnp.bfloat16)
    bf = jnp.concatenate([b3, b5, b7], axis=0).reshape(1, CT).astype(jnp.float32)
    ow8 = jnp.broadcast_to(ow.reshape(1, CT).astype(jnp.float32), (8, CT))
    ob1 = ob.reshape(1).astype(jnp.float32)

    Hp, Wp, K7 = H + 2 * pad, W + 2 * pad + 2, k * Cin

    flops = (2 * N * HW * (49 * Cin * 2 * _LANE + 9 * Cin * _LANE)
             + 2 * N * HW * CT)
    cost = pl.CostEstimate(
        flops=flops,
        transcendentals=0,
        bytes_accessed=(x_up.size * 2 + wb.size * 2 + w3m.size * 2
                        + bf.size * 4 + ow8.size * 4 + N * HW * 4),
    )
    IPB = 1
    fn = functools.partial(_heads_kernel, H=H, W=W, Cin=Cin, IPB=IPB)
    out = pl.pallas_call(
        fn,
        out_shape=jax.ShapeDtypeStruct((N // IPB, 1, IPB * HW), jnp.float32),
        grid_spec=pltpu.PrefetchScalarGridSpec(
            num_scalar_prefetch=0,
            grid=(N // IPB,),
            in_specs=[
                pl.BlockSpec((IPB, H, W, Cin), lambda n: (n, 0, 0, 0)),
                pl.BlockSpec((7, K7, 2 * _LANE), lambda n: (0, 0, 0),
                             pipeline_mode=_RESIDENT),
                pl.BlockSpec((3, 3 * Cin, _LANE), lambda n: (0, 0, 0),
                             pipeline_mode=_RESIDENT),
                pl.BlockSpec((1, CT), lambda n: (0, 0), pipeline_mode=_RESIDENT),
                pl.BlockSpec((8, CT), lambda n: (0, 0), pipeline_mode=_RESIDENT),
                pl.BlockSpec(memory_space=pltpu.MemorySpace.SMEM),
            ],
            out_specs=pl.BlockSpec((1, 1, IPB * HW), lambda n: (n, 0, 0)),
            scratch_shapes=[
                pltpu.VMEM((Hp, Wp, Cin), jnp.bfloat16),
                pltpu.VMEM((Hp, IPB * W, K7), jnp.bfloat16),
                pltpu.VMEM((IPB * HW, 2 * _LANE), jnp.float32),
                pltpu.VMEM((IPB * HW, _LANE), jnp.float32),
            ],
        ),
        compiler_params=_cparams(("parallel",)),
        cost_estimate=cost,
    )(x_up, wb, w3m, bf, ow8, ob1)
    out = out.reshape(N // IPB, H, IPB, W).transpose(0, 2, 1, 3)
    return out.reshape(N, 1, H, W)


def kernel(x_nchw, features_w, features_b, reg3_w, reg3_b, reg5_w, reg5_b,
           reg7_w, reg7_b, out_w, out_b):
    x = _trunk(x_nchw, features_w, features_b)
    x = _upsample2x(x)
    return _heads(x, reg3_w, reg3_b, reg5_w, reg5_b, reg7_w, reg7_b,
                  out_w, out_b)
